# baseline (speedup 1.0000x reference)
import functools

import jax
import jax.numpy as jnp
from jax import lax
from jax.experimental import pallas as pl
from jax.experimental.pallas import tpu as pltpu
from jax.experimental.pallas import tpu_sc as plsc

_BATCH = 4096
_FEATS_PER_RANK = (7, 7, 6, 6)
_NC, _NS = 2, 16
_NW = _NC * _NS

_BOUNDS = [0]
for _f in _FEATS_PER_RANK:
    _BOUNDS.append(_BOUNDS[-1] + _f * _BATCH)

_JOBS = []
_OUT_TYPE = []
for _r in range(4):
    _b0, _b1 = _BOUNDS[_r], _BOUNDS[_r + 1]
    _sz = _b1 - _b0
    for _slot, _off, _words, _dt in (
        (0, 2 * _b0, 2 * _sz, jnp.int32),
        (1, _b0, _sz, jnp.int32),
        (2, 2 * _b0, 2 * _sz, jnp.int32),
        (3, _b0, _sz, jnp.float32),
        (4, _b0, _sz, jnp.int32),
    ):
        _JOBS.append((_slot, _off, _words))
        _OUT_TYPE.append(jax.ShapeDtypeStruct((_words,), _dt))


@functools.partial(
    pl.kernel,
    mesh=plsc.VectorSubcoreMesh(core_axis_name="c", subcore_axis_name="s"),
    out_type=_OUT_TYPE,
    scratch_types=[pltpu.SemaphoreType.DMA],
)
def _split_sc(v_in, l_in, sv_in, w_in, sl_in, *outs_and_sem):
    outs, sem = outs_and_sem[:-1], outs_and_sem[-1]
    ins = (v_in, l_in, sv_in, w_in, sl_in)
    wid = lax.axis_index("s") * _NC + lax.axis_index("c")
    descs = []
    for (slot, off, words), out in zip(_JOBS, outs):
        chunk = words // _NW
        o = pl.multiple_of(wid * chunk, 8)
        descs.append(
            pltpu.async_copy(
                ins[slot].at[pl.ds(off + o, chunk)],
                out.at[pl.ds(o, chunk)],
                sem,
            )
        )
    for d in descs:
        d.wait()


def kernel(id_list_values, id_list_lengths, id_score_list_values,
           id_score_list_weights, id_score_list_lengths):
    v32 = lax.bitcast_convert_type(id_list_values, jnp.int32).reshape(-1)
    sv32 = lax.bitcast_convert_type(id_score_list_values, jnp.int32).reshape(-1)
    res = _split_sc(v32, id_list_lengths, sv32,
                    id_score_list_weights, id_score_list_lengths)
    outs = []
    for r in range(4):
        v, l, sv, w, sl = res[5 * r:5 * r + 5]
        outs.append(lax.bitcast_convert_type(v.reshape(-1, 2), jnp.int64))
        outs.append(l)
        outs.append(lax.bitcast_convert_type(sv.reshape(-1, 2), jnp.int64))
        outs.append(w)
        outs.append(sl)
    return tuple(outs)
